# baseline (speedup 1.0000x reference)
import functools

import jax
import jax.numpy as jnp
from jax import lax
from jax.experimental import pallas as pl
from jax.experimental.pallas import tpu as pltpu
from jax.experimental.pallas import tpu_sc as plsc

_PATCH = 14.0
_THR2 = 100.00001
_NPAD = 1024
_BI = 128
_NF = 5


def _rne_bf16(x):
    u = jax.lax.bitcast_convert_type(x, jnp.int32)
    tie = jax.lax.shift_right_logical(u, 16) & 1
    u = u + jnp.int32(0x7FFF) + tie
    u = jnp.bitwise_and(u, jnp.int32(~0xFFFF))
    return jax.lax.bitcast_convert_type(u, jnp.float32)


def _score_body(n_actual, rows_ref, cols_ref, meta_ref, mask_ref):
    f32 = jnp.float32
    r = rows_ref[0]
    px = r[0:1, :]
    py = r[1:2, :]
    qx = r[2:3, :]
    qy = r[3:4, :]
    w = r[4:5, :]
    psx = _rne_bf16(px * _PATCH)
    psy = _rne_bf16(py * _PATCH)
    qsx = qx * _PATCH
    qsy = qy * _PATCH
    jlane = jax.lax.broadcasted_iota(jnp.int32, (1, _NPAD), 1)
    wrow = jnp.where(jlane < n_actual, w, 0.0)

    cc = cols_ref[0]
    pxt = cc[:, 0:1]
    pyt = cc[:, 1:2]
    qxt = cc[:, 2:3]
    qyt = cc[:, 3:4]
    wt = cc[:, 4:5]
    st = cc[:, 5:6]
    ct = cc[:, 6:7]
    snt = cc[:, 7:8]
    psxt = _rne_bf16(pxt * _PATCH)
    psyt = _rne_bf16(pyt * _PATCH)
    a_c = st * ct
    b_c = st * snt
    a16 = _rne_bf16(a_c)
    b16 = _rne_bf16(b_c)
    tx_c = qxt * _PATCH - (a16 * psxt - b16 * psyt)
    ty_c = qyt * _PATCH - (b16 * psxt + a16 * psyt)
    icol = jax.lax.broadcasted_iota(jnp.int32, (_NPAD, 1), 0)

    nblk = _NPAD // _BI
    score_cols = []
    for ib in range(nblk):
        sl = slice(ib * _BI, (ib + 1) * _BI)
        ac = a16[sl]
        bc = b16[sl]
        txc = tx_c[sl]
        tyc = ty_c[sl]
        predx = ac * psx - bc * psy + txc
        predy = bc * psx + ac * psy + tyc
        ex = qsx - predx
        ey = qsy - predy
        e2 = ex * ex + ey * ey
        contrib = jnp.where(e2 <= _THR2, wrow, 0.0)
        srow = jnp.sum(contrib, axis=1, keepdims=True) - wt[sl]
        srow = jnp.where(icol[sl] < n_actual, srow, -1.0)
        score_cols.append(srow)
    score = jnp.concatenate(score_cols, axis=1)

    best_score = jnp.max(score)
    rix = jax.lax.broadcasted_iota(jnp.int32, (_BI, nblk), 0)
    cix = jax.lax.broadcasted_iota(jnp.int32, (_BI, nblk), 1)
    iidx = cix * _BI + rix
    best_i = jnp.min(jnp.where(score == best_score, iidx, jnp.int32(1 << 30)))

    sel = icol == best_i
    a_b = jnp.sum(jnp.where(sel, a_c, 0.0))
    b_b = jnp.sum(jnp.where(sel, b_c, 0.0))
    a16_b = jnp.sum(jnp.where(sel, a16, 0.0))
    b16_b = jnp.sum(jnp.where(sel, b16, 0.0))
    tx_b = jnp.sum(jnp.where(sel, tx_c, 0.0))
    ty_b = jnp.sum(jnp.where(sel, ty_c, 0.0))

    predxb = a16_b * psx - b16_b * psy + tx_b
    predyb = b16_b * psx + a16_b * psy + ty_b
    exb = qsx - predxb
    eyb = qsy - predyb
    e2b = exb * exb + eyb * eyb
    mask = (e2b <= _THR2) & (jlane < n_actual) & (jlane != best_i)
    mask_ref[0] = mask.astype(jnp.int32)

    r8 = jax.lax.broadcasted_iota(jnp.int32, (8, 128), 0)
    c1 = jax.lax.broadcasted_iota(jnp.int32, (8, 128), 1)
    mv = jnp.where((r8 == 2) & (c1 == 2), 1.0, 0.0)
    mv = jnp.where((r8 == 0) & (c1 == 0), a_b, mv)
    mv = jnp.where((r8 == 0) & (c1 == 1), -b_b, mv)
    mv = jnp.where((r8 == 0) & (c1 == 2), tx_b, mv)
    mv = jnp.where((r8 == 0) & (c1 == 3), best_score, mv)
    mv = jnp.where((r8 == 1) & (c1 == 0), b_b, mv)
    mv = jnp.where((r8 == 1) & (c1 == 1), a_b, mv)
    mv = jnp.where((r8 == 1) & (c1 == 2), ty_b, mv)
    meta_ref[0] = mv


def _compact_body(nb, n_actual, rows_hbm, mask_hbm,
                  isrc_hbm, itar_hbm, iscr_hbm,
                  mask_v, feat_v, idx_v, osrc_v, otar_v, oscr_v):
    c = lax.axis_index("c")
    s = lax.axis_index("s")
    wid = s * 2 + c

    @pl.when(wid < nb)
    def _():
        b = wid
        pltpu.sync_copy(mask_hbm.at[pl.ds(b * _NPAD, _NPAD)], mask_v)
        pltpu.sync_copy(rows_hbm.at[pl.ds(b * 8 * _NPAD, _NF * _NPAD)], feat_v)

        nchunk = _NPAD // 16

        def pass1(k, carry):
            off = k * 16
            mv = mask_v[pl.ds(off, 16)]
            m = mv != 0
            mi = jnp.where(m, jnp.int32(1), jnp.int32(0))
            pos = plsc.cumsum(mi)
            jv = off + lax.iota(jnp.int32, 16)
            tgt = jnp.maximum(pos + carry - 1, 0)
            plsc.store_scatter(idx_v, [tgt], jv, mask=m)
            return carry + jnp.sum(mi)

        count = lax.fori_loop(0, nchunk, pass1, jnp.int32(0))

        def pass2(k, dummy):
            off = k * 16
            kv = off + lax.iota(jnp.int32, 16)
            valid = kv < count
            idxs = jnp.where(valid, idx_v[pl.ds(off, 16)], jnp.int32(0))
            gx = plsc.load_gather(feat_v, [idxs])
            gy = plsc.load_gather(feat_v, [idxs + _NPAD])
            hx = plsc.load_gather(feat_v, [idxs + 2 * _NPAD])
            hy = plsc.load_gather(feat_v, [idxs + 3 * _NPAD])
            gw = plsc.load_gather(feat_v, [idxs + 4 * _NPAD])
            kv2 = kv * 2
            plsc.store_scatter(osrc_v, [kv2], jnp.where(valid, gx, -1.0))
            plsc.store_scatter(osrc_v, [kv2 + 1], jnp.where(valid, gy, -1.0))
            plsc.store_scatter(otar_v, [kv2], jnp.where(valid, hx, -1.0))
            plsc.store_scatter(otar_v, [kv2 + 1], jnp.where(valid, hy, -1.0))
            oscr_v[pl.ds(off, 16)] = jnp.where(valid, gw, 0.0)
            return dummy

        lax.fori_loop(0, nchunk, pass2, jnp.int32(0))
        n2 = 2 * n_actual
        pltpu.sync_copy(osrc_v.at[pl.ds(0, n2)], isrc_hbm.at[pl.ds(b * n2, n2)])
        pltpu.sync_copy(otar_v.at[pl.ds(0, n2)], itar_hbm.at[pl.ds(b * n2, n2)])
        pltpu.sync_copy(oscr_v.at[pl.ds(0, n_actual)],
                        iscr_hbm.at[pl.ds(b * n_actual, n_actual)])


def kernel(src_pts, tar_pts, relScales, relInplanes, scores):
    B, N = src_pts.shape[:2]
    f32 = jnp.float32
    pad = _NPAD - N

    feats = jnp.stack([src_pts[..., 0], src_pts[..., 1],
                       tar_pts[..., 0], tar_pts[..., 1],
                       scores, relScales,
                       relInplanes[..., 0], relInplanes[..., 1]], axis=1)
    rows = jnp.pad(feats, ((0, 0), (0, 0), (0, pad)))
    cols = jnp.swapaxes(rows, 1, 2)

    meta, mask = pl.pallas_call(
        functools.partial(_score_body, N),
        grid=(B,),
        in_specs=[pl.BlockSpec((1, 8, _NPAD), lambda b: (b, 0, 0)),
                  pl.BlockSpec((1, _NPAD, 8), lambda b: (b, 0, 0))],
        out_specs=[pl.BlockSpec((1, 8, 128), lambda b: (b, 0, 0)),
                   pl.BlockSpec((1, 1, _NPAD), lambda b: (b, 0, 0))],
        out_shape=[jax.ShapeDtypeStruct((B, 8, 128), f32),
                   jax.ShapeDtypeStruct((B, 1, _NPAD), jnp.int32)],
    )(rows, cols)

    mesh = plsc.VectorSubcoreMesh(core_axis_name="c", subcore_axis_name="s",
                                  num_cores=2)
    compact = pl.kernel(
        functools.partial(_compact_body, B, N),
        mesh=mesh,
        out_type=[jax.ShapeDtypeStruct((B * 2 * N,), f32),
                  jax.ShapeDtypeStruct((B * 2 * N,), f32),
                  jax.ShapeDtypeStruct((B * N,), f32)],
        scratch_types=[pltpu.VMEM((_NPAD,), jnp.int32),
                       pltpu.VMEM((_NF * _NPAD,), f32),
                       pltpu.VMEM((_NPAD,), jnp.int32),
                       pltpu.VMEM((2 * _NPAD,), f32),
                       pltpu.VMEM((2 * _NPAD,), f32),
                       pltpu.VMEM((_NPAD,), f32)],
        compiler_params=pltpu.CompilerParams(needs_layout_passes=False),
    )
    isrc_f, itar_f, iscr_f = compact(rows.reshape(B * 8 * _NPAD),
                                     mask.reshape(B * _NPAD))

    M = meta[:, :3, :3]
    failed = meta[:, 0, 3] == 0.0
    isrc = isrc_f.reshape(B, N, 2)
    itar = itar_f.reshape(B, N, 2)
    iscr = iscr_f.reshape(B, N)
    return M, failed, isrc, itar, iscr


# speedup vs baseline: 1.1494x; 1.1494x over previous
import functools

import jax
import jax.numpy as jnp
from jax import lax
from jax.experimental import pallas as pl
from jax.experimental.pallas import tpu as pltpu
from jax.experimental.pallas import tpu_sc as plsc

_PATCH = 14.0
_THR2 = 100.00001
_NPAD = 1024
_BI = 128
_NF = 5


def _rne_bf16(x):
    u = jax.lax.bitcast_convert_type(x, jnp.int32)
    tie = jax.lax.shift_right_logical(u, 16) & 1
    u = u + jnp.int32(0x7FFF) + tie
    u = jnp.bitwise_and(u, jnp.int32(~0xFFFF))
    return jax.lax.bitcast_convert_type(u, jnp.float32)


def _score_body(n_actual, rows_ref, cols_ref, meta_ref, mask_ref):
    f32 = jnp.float32
    r = rows_ref[0]
    px = r[0:1, :]
    py = r[1:2, :]
    qx = r[2:3, :]
    qy = r[3:4, :]
    w = r[4:5, :]
    psx = _rne_bf16(px * _PATCH)
    psy = _rne_bf16(py * _PATCH)
    qsx = qx * _PATCH
    qsy = qy * _PATCH
    jlane = jax.lax.broadcasted_iota(jnp.int32, (1, _NPAD), 1)
    wrow = jnp.where(jlane < n_actual, w, 0.0)

    cc = cols_ref[0]
    pxt = cc[:, 0:1]
    pyt = cc[:, 1:2]
    qxt = cc[:, 2:3]
    qyt = cc[:, 3:4]
    wt = cc[:, 4:5]
    st = cc[:, 5:6]
    ct = cc[:, 6:7]
    snt = cc[:, 7:8]
    psxt = _rne_bf16(pxt * _PATCH)
    psyt = _rne_bf16(pyt * _PATCH)
    a_c = st * ct
    b_c = st * snt
    a16 = _rne_bf16(a_c)
    b16 = _rne_bf16(b_c)
    tx_c = qxt * _PATCH - (a16 * psxt - b16 * psyt)
    ty_c = qyt * _PATCH - (b16 * psxt + a16 * psyt)
    icol = jax.lax.broadcasted_iota(jnp.int32, (_NPAD, 1), 0)

    nblk = _NPAD // _BI
    score_cols = []
    for ib in range(nblk):
        sl = slice(ib * _BI, (ib + 1) * _BI)
        ac = a16[sl]
        bc = b16[sl]
        txc = tx_c[sl]
        tyc = ty_c[sl]
        predx = ac * psx - bc * psy + txc
        predy = bc * psx + ac * psy + tyc
        ex = qsx - predx
        ey = qsy - predy
        e2 = ex * ex + ey * ey
        contrib = jnp.where(e2 <= _THR2, wrow, 0.0)
        srow = jnp.sum(contrib, axis=1, keepdims=True) - wt[sl]
        srow = jnp.where(icol[sl] < n_actual, srow, -1.0)
        score_cols.append(srow)
    score = jnp.concatenate(score_cols, axis=1)

    best_score = jnp.max(score)
    rix = jax.lax.broadcasted_iota(jnp.int32, (_BI, nblk), 0)
    cix = jax.lax.broadcasted_iota(jnp.int32, (_BI, nblk), 1)
    iidx = cix * _BI + rix
    best_i = jnp.min(jnp.where(score == best_score, iidx, jnp.int32(1 << 30)))

    sel = icol == best_i
    a_b = jnp.sum(jnp.where(sel, a_c, 0.0))
    b_b = jnp.sum(jnp.where(sel, b_c, 0.0))
    a16_b = jnp.sum(jnp.where(sel, a16, 0.0))
    b16_b = jnp.sum(jnp.where(sel, b16, 0.0))
    tx_b = jnp.sum(jnp.where(sel, tx_c, 0.0))
    ty_b = jnp.sum(jnp.where(sel, ty_c, 0.0))

    predxb = a16_b * psx - b16_b * psy + tx_b
    predyb = b16_b * psx + a16_b * psy + ty_b
    exb = qsx - predxb
    eyb = qsy - predyb
    e2b = exb * exb + eyb * eyb
    mask = (e2b <= _THR2) & (jlane < n_actual) & (jlane != best_i)
    mask_ref[0] = mask.astype(jnp.int32)

    r8 = jax.lax.broadcasted_iota(jnp.int32, (8, 128), 0)
    c1 = jax.lax.broadcasted_iota(jnp.int32, (8, 128), 1)
    mv = jnp.where((r8 == 2) & (c1 == 2), 1.0, 0.0)
    mv = jnp.where((r8 == 0) & (c1 == 0), a_b, mv)
    mv = jnp.where((r8 == 0) & (c1 == 1), -b_b, mv)
    mv = jnp.where((r8 == 0) & (c1 == 2), tx_b, mv)
    mv = jnp.where((r8 == 0) & (c1 == 3), best_score, mv)
    mv = jnp.where((r8 == 1) & (c1 == 0), b_b, mv)
    mv = jnp.where((r8 == 1) & (c1 == 1), a_b, mv)
    mv = jnp.where((r8 == 1) & (c1 == 2), ty_b, mv)
    meta_ref[0] = mv


def _compact_body(nb, rows_hbm, mask_hbm, out_hbm,
                  mask_v, feat_v, idx_v, out_v):
    c = lax.axis_index("c")
    s = lax.axis_index("s")
    wid = s * 2 + c

    @pl.when(wid < nb)
    def _():
        b = wid
        pltpu.sync_copy(mask_hbm.at[pl.ds(b * _NPAD, _NPAD)], mask_v)
        pltpu.sync_copy(rows_hbm.at[pl.ds(b * 8 * _NPAD, _NF * _NPAD)], feat_v)

        nchunk = _NPAD // 16

        def pass1(k, carry):
            off = k * 16
            mv = mask_v[pl.ds(off, 16)]
            m = mv != 0
            mi = jnp.where(m, jnp.int32(1), jnp.int32(0))
            pos = plsc.cumsum(mi)
            jv = off + lax.iota(jnp.int32, 16)
            tgt = jnp.maximum(pos + carry - 1, 0)
            plsc.store_scatter(idx_v, [tgt], jv, mask=m)
            return carry + jnp.sum(mi)

        count = lax.fori_loop(0, nchunk, pass1, jnp.int32(0))

        def pass2(k, dummy):
            off = k * 16
            kv = off + lax.iota(jnp.int32, 16)
            valid = kv < count
            idxs = jnp.where(valid, idx_v[pl.ds(off, 16)], jnp.int32(0))
            for r in range(_NF):
                g = plsc.load_gather(feat_v, [idxs + r * _NPAD])
                fill = -1.0 if r < 4 else 0.0
                out_v[pl.ds(off + r * _NPAD, 16)] = jnp.where(valid, g, fill)
            return dummy

        lax.fori_loop(0, nchunk, pass2, jnp.int32(0))
        pltpu.sync_copy(out_v, out_hbm.at[pl.ds(b * _NF * _NPAD, _NF * _NPAD)])


def kernel(src_pts, tar_pts, relScales, relInplanes, scores):
    B, N = src_pts.shape[:2]
    f32 = jnp.float32
    pad = _NPAD - N

    feats = jnp.stack([src_pts[..., 0], src_pts[..., 1],
                       tar_pts[..., 0], tar_pts[..., 1],
                       scores, relScales,
                       relInplanes[..., 0], relInplanes[..., 1]], axis=1)
    rows = jnp.pad(feats, ((0, 0), (0, 0), (0, pad)))
    cols = jnp.swapaxes(rows, 1, 2)

    meta, mask = pl.pallas_call(
        functools.partial(_score_body, N),
        grid=(B,),
        in_specs=[pl.BlockSpec((1, 8, _NPAD), lambda b: (b, 0, 0)),
                  pl.BlockSpec((1, _NPAD, 8), lambda b: (b, 0, 0))],
        out_specs=[pl.BlockSpec((1, 8, 128), lambda b: (b, 0, 0)),
                   pl.BlockSpec((1, 1, _NPAD), lambda b: (b, 0, 0))],
        out_shape=[jax.ShapeDtypeStruct((B, 8, 128), f32),
                   jax.ShapeDtypeStruct((B, 1, _NPAD), jnp.int32)],
    )(rows, cols)

    mesh = plsc.VectorSubcoreMesh(core_axis_name="c", subcore_axis_name="s",
                                  num_cores=2)
    compact = pl.kernel(
        functools.partial(_compact_body, B),
        mesh=mesh,
        out_type=jax.ShapeDtypeStruct((B * _NF * _NPAD,), f32),
        scratch_types=[pltpu.VMEM((_NPAD,), jnp.int32),
                       pltpu.VMEM((_NF * _NPAD,), f32),
                       pltpu.VMEM((_NPAD,), jnp.int32),
                       pltpu.VMEM((_NF * _NPAD,), f32)],
        compiler_params=pltpu.CompilerParams(needs_layout_passes=False),
    )
    packed = compact(rows.reshape(B * 8 * _NPAD), mask.reshape(B * _NPAD))
    packed = packed.reshape(B, _NF, _NPAD)

    M = meta[:, :3, :3]
    failed = meta[:, 0, 3] == 0.0
    isrc = jnp.stack([packed[:, 0, :N], packed[:, 1, :N]], axis=-1)
    itar = jnp.stack([packed[:, 2, :N], packed[:, 3, :N]], axis=-1)
    iscr = packed[:, 4, :N]
    return M, failed, isrc, itar, iscr
